# baseline (device time: 22371 ns/iter reference)
import jax
import jax.numpy as jnp
from jax import lax
from jax.experimental import pallas as pl
from jax.experimental.pallas import tpu as pltpu

N_DEV = 8
STEPS = (1, 2, 4)


def kernel(x, router_W, route_idx, expert_W, shared_W):
    n_tok, d_model = x.shape
    n_exp = router_W.shape[1]
    e_local, _, d_h = expert_W.shape

    def body(x_ref, rw_ref, idx_ref, ew_ref, sw_ref, out_ref,
             comm_ref, send_sems, recv_sems):
        me = lax.axis_index("i")

        barrier = pltpu.get_barrier_semaphore()
        for k in STEPS:
            pl.semaphore_signal(
                barrier, inc=1,
                device_id=(me ^ k,), device_id_type=pl.DeviceIdType.MESH,
            )
        pl.semaphore_wait(barrier, len(STEPS))

        x_f32 = x_ref[:, :]
        scores = jnp.dot(x_f32, rw_ref[:, :],
                         preferred_element_type=jnp.float32)
        s_max = jnp.max(scores, axis=-1, keepdims=True)
        p = jnp.exp(scores - s_max)
        probs = p / jnp.sum(p, axis=-1, keepdims=True)

        x_bf = x_f32.astype(jnp.bfloat16)
        idx = idx_ref[:, :]
        exp_iota = lax.broadcasted_iota(jnp.int32, (n_tok, n_exp), 1)
        partial = jnp.zeros((n_tok, d_h), jnp.float32)
        for le in range(e_local):
            e = e_local * me + le
            y = jnp.dot(x_bf, ew_ref[le].astype(jnp.bfloat16),
                        preferred_element_type=jnp.float32)
            prob_e = jnp.sum(probs * (exp_iota == e).astype(jnp.float32),
                             axis=-1, keepdims=True)
            gate = jnp.where(idx == e, prob_e, 0.0)
            partial = partial + gate * y
        out_ref[:, :] = partial

        for step, k in enumerate(STEPS):
            rdma = pltpu.make_async_remote_copy(
                src_ref=out_ref,
                dst_ref=comm_ref.at[step],
                send_sem=send_sems.at[step],
                recv_sem=recv_sems.at[step],
                device_id=(me ^ k,),
                device_id_type=pl.DeviceIdType.MESH,
            )
            rdma.start()
            rdma.wait()
            out_ref[:, :] = out_ref[:, :] + comm_ref[step]

        shared = jnp.dot(x_bf, sw_ref[:, :].astype(jnp.bfloat16),
                         preferred_element_type=jnp.float32)
        out_ref[:, :] = out_ref[:, :] + shared

    return pl.pallas_call(
        body,
        out_shape=jax.ShapeDtypeStruct((n_tok, d_h), jnp.float32),
        in_specs=[pl.BlockSpec(memory_space=pltpu.VMEM)] * 5,
        out_specs=pl.BlockSpec(memory_space=pltpu.VMEM),
        scratch_shapes=[
            pltpu.VMEM((len(STEPS), n_tok, d_h), jnp.float32),
            pltpu.SemaphoreType.DMA((len(STEPS),)),
            pltpu.SemaphoreType.DMA((len(STEPS),)),
        ],
        compiler_params=pltpu.CompilerParams(collective_id=0),
    )(x, router_W, route_idx, expert_W, shared_W)


# device time: 17143 ns/iter; 1.3050x vs baseline; 1.3050x over previous
import jax
import jax.numpy as jnp
from jax import lax
from jax.experimental import pallas as pl
from jax.experimental.pallas import tpu as pltpu

N_DEV = 8
STEPS = (1, 3, 4)


def kernel(x, router_W, route_idx, expert_W, shared_W):
    n_tok, d_model = x.shape
    n_exp = router_W.shape[1]
    e_local, _, d_h = expert_W.shape

    def body(x_ref, rw_ref, idx_ref, ew_ref, sw_ref, out_ref,
             comm_ref, send_buf, send_sems, recv_sems):
        me = lax.axis_index("i")

        barrier = pltpu.get_barrier_semaphore()
        for k in STEPS:
            pl.semaphore_signal(
                barrier, inc=1,
                device_id=(me ^ k,), device_id_type=pl.DeviceIdType.MESH,
            )
        pl.semaphore_wait(barrier, len(STEPS))

        x_f32 = x_ref[:, :]
        scores = jnp.dot(x_f32, rw_ref[:, :],
                         preferred_element_type=jnp.float32)
        s_max = jnp.max(scores, axis=-1, keepdims=True)
        p = jnp.exp(scores - s_max)
        probs = p / jnp.sum(p, axis=-1, keepdims=True)

        x_bf = x_f32.astype(jnp.bfloat16)
        idx = idx_ref[:, :]
        exp_iota = lax.broadcasted_iota(jnp.int32, (n_tok, n_exp), 1)
        partial = jnp.zeros((n_tok, d_h), jnp.float32)
        for le in range(e_local):
            e = e_local * me + le
            y = jnp.dot(x_bf, ew_ref[le].astype(jnp.bfloat16),
                        preferred_element_type=jnp.float32)
            prob_e = jnp.sum(probs * (exp_iota == e).astype(jnp.float32),
                             axis=-1, keepdims=True)
            gate = jnp.where(idx == e, prob_e, 0.0)
            partial = partial + gate * y

        acc = partial
        shared = None
        for step, k in enumerate(STEPS):
            send_buf[step] = acc.astype(jnp.bfloat16)
            rdma = pltpu.make_async_remote_copy(
                src_ref=send_buf.at[step],
                dst_ref=comm_ref.at[step],
                send_sem=send_sems.at[step],
                recv_sem=recv_sems.at[step],
                device_id=(me ^ k,),
                device_id_type=pl.DeviceIdType.MESH,
            )
            rdma.start()
            if step == 0:
                shared = jnp.dot(x_bf, sw_ref[:, :].astype(jnp.bfloat16),
                                 preferred_element_type=jnp.float32)
            rdma.wait()
            acc = acc + comm_ref[step].astype(jnp.float32)

        out_ref[:, :] = acc + shared

    return pl.pallas_call(
        body,
        out_shape=jax.ShapeDtypeStruct((n_tok, d_h), jnp.float32),
        in_specs=[pl.BlockSpec(memory_space=pltpu.VMEM)] * 5,
        out_specs=pl.BlockSpec(memory_space=pltpu.VMEM),
        scratch_shapes=[
            pltpu.VMEM((len(STEPS), n_tok, d_h), jnp.bfloat16),
            pltpu.VMEM((len(STEPS), n_tok, d_h), jnp.bfloat16),
            pltpu.SemaphoreType.DMA((len(STEPS),)),
            pltpu.SemaphoreType.DMA((len(STEPS),)),
        ],
        compiler_params=pltpu.CompilerParams(collective_id=0),
    )(x, router_W, route_idx, expert_W, shared_W)


# device time: 15728 ns/iter; 1.4224x vs baseline; 1.0900x over previous
import jax
import jax.numpy as jnp
from jax import lax
from jax.experimental import pallas as pl
from jax.experimental.pallas import tpu as pltpu

N_DEV = 8
STEPS = (1, 3, 4)


def kernel(x, router_W, route_idx, expert_W, shared_W):
    n_tok, d_model = x.shape
    n_exp = router_W.shape[1]
    e_local, _, d_h = expert_W.shape

    def body(x_ref, rw_ref, idx_ref, ew_ref, sw_ref, out_ref,
             comm_ref, send_buf, send_sems, recv_sems):
        me = lax.axis_index("i")

        barrier = pltpu.get_barrier_semaphore()
        for k in STEPS:
            pl.semaphore_signal(
                barrier, inc=1,
                device_id=(me ^ k,), device_id_type=pl.DeviceIdType.MESH,
            )
        pl.semaphore_wait(barrier, len(STEPS))

        x_f32 = x_ref[:, :]
        scores = jnp.dot(x_f32, rw_ref[:, :],
                         preferred_element_type=jnp.float32)
        s_max = jnp.max(scores, axis=-1, keepdims=True)
        p = jnp.exp(scores - s_max)
        probs = p / jnp.sum(p, axis=-1, keepdims=True)

        x_bf = x_f32.astype(jnp.bfloat16)
        idx = idx_ref[:, :]
        exp_iota = lax.broadcasted_iota(jnp.int32, (n_tok, n_exp), 1)
        w_bf = ew_ref[:, :, :].astype(jnp.bfloat16)

        hm = n_tok // 2

        def expert_partial(r0):
            xs = x_bf[r0:r0 + hm]
            part = jnp.zeros((hm, d_h), jnp.float32)
            for le in range(e_local):
                e = e_local * me + le
                y = jnp.dot(xs, w_bf[le], preferred_element_type=jnp.float32)
                prob_e = jnp.sum(
                    probs[r0:r0 + hm] *
                    (exp_iota[r0:r0 + hm] == e).astype(jnp.float32),
                    axis=-1, keepdims=True)
                gate = jnp.where(idx[r0:r0 + hm] == e, prob_e, 0.0)
                part = part + gate * y
            return part

        def make_rdma(step, half, k):
            return pltpu.make_async_remote_copy(
                src_ref=send_buf.at[step, half],
                dst_ref=comm_ref.at[step, half],
                send_sem=send_sems.at[step, half],
                recv_sem=recv_sems.at[step, half],
                device_id=(me ^ k,),
                device_id_type=pl.DeviceIdType.MESH,
            )

        acc = [None, None]
        acc[0] = expert_partial(0)
        send_buf[0, 0] = acc[0].astype(jnp.bfloat16)
        rdmas = [[None] * 2 for _ in STEPS]
        rdmas[0][0] = make_rdma(0, 0, STEPS[0])
        rdmas[0][0].start()

        acc[1] = expert_partial(hm)
        send_buf[0, 1] = acc[1].astype(jnp.bfloat16)
        rdmas[0][1] = make_rdma(0, 1, STEPS[0])
        rdmas[0][1].start()

        shared = jnp.dot(x_bf, sw_ref[:, :].astype(jnp.bfloat16),
                         preferred_element_type=jnp.float32)

        for step, k in enumerate(STEPS):
            for half in range(2):
                rdmas[step][half].wait()
                acc[half] = acc[half] + comm_ref[step, half].astype(jnp.float32)
                if step + 1 < len(STEPS):
                    send_buf[step + 1, half] = acc[half].astype(jnp.bfloat16)
                    rdmas[step + 1][half] = make_rdma(step + 1, half,
                                                      STEPS[step + 1])
                    rdmas[step + 1][half].start()

        out_ref[0:hm, :] = acc[0] + shared[0:hm]
        out_ref[hm:n_tok, :] = acc[1] + shared[hm:n_tok]

    return pl.pallas_call(
        body,
        out_shape=jax.ShapeDtypeStruct((n_tok, d_h), jnp.float32),
        in_specs=[pl.BlockSpec(memory_space=pltpu.VMEM)] * 5,
        out_specs=pl.BlockSpec(memory_space=pltpu.VMEM),
        scratch_shapes=[
            pltpu.VMEM((len(STEPS), 2, n_tok // 2, d_h), jnp.bfloat16),
            pltpu.VMEM((len(STEPS), 2, n_tok // 2, d_h), jnp.bfloat16),
            pltpu.SemaphoreType.DMA((len(STEPS), 2)),
            pltpu.SemaphoreType.DMA((len(STEPS), 2)),
        ],
        compiler_params=pltpu.CompilerParams(collective_id=0),
    )(x, router_W, route_idx, expert_W, shared_W)


# device time: 15510 ns/iter; 1.4424x vs baseline; 1.0141x over previous
import jax
import jax.numpy as jnp
from jax import lax
from jax.experimental import pallas as pl
from jax.experimental.pallas import tpu as pltpu

N_DEV = 8
STEPS = (1, 3, 4)


def kernel(x, router_W, route_idx, expert_W, shared_W):
    n_tok, d_model = x.shape
    n_exp = router_W.shape[1]
    e_local, _, d_h = expert_W.shape

    def body(x_ref, rw_ref, idx_ref, ew_ref, sw_ref, out_ref,
             comm_ref, send_buf, send_sems, recv_sems):
        me = lax.axis_index("i")

        import os
        skip_barrier = os.environ.get("SKIP_BARRIER") == "1"

        if not skip_barrier:
            barrier = pltpu.get_barrier_semaphore()
            for k in STEPS:
                pl.semaphore_signal(
                    barrier, inc=1,
                    device_id=(me ^ k,), device_id_type=pl.DeviceIdType.MESH,
                )

        x_f32 = x_ref[:, :]
        scores = jnp.dot(x_f32, rw_ref[:, :],
                         preferred_element_type=jnp.float32)
        s_max = jnp.max(scores, axis=-1, keepdims=True)
        p = jnp.exp(scores - s_max)
        probs = p / jnp.sum(p, axis=-1, keepdims=True)

        x_bf = x_f32.astype(jnp.bfloat16)
        idx = idx_ref[:, :]
        exp_iota = lax.broadcasted_iota(jnp.int32, (n_tok, n_exp), 1)
        w_bf = ew_ref[:, :, :].astype(jnp.bfloat16)

        hm = n_tok // 2

        w_cat = jnp.concatenate([w_bf[le] for le in range(e_local)], axis=1)

        def expert_partial(r0):
            xs = x_bf[r0:r0 + hm]
            y = jnp.dot(xs, w_cat, preferred_element_type=jnp.float32)
            part = jnp.zeros((hm, d_h), jnp.float32)
            for le in range(e_local):
                e = e_local * me + le
                prob_e = jnp.sum(
                    probs[r0:r0 + hm] *
                    (exp_iota[r0:r0 + hm] == e).astype(jnp.float32),
                    axis=-1, keepdims=True)
                gate = jnp.where(idx[r0:r0 + hm] == e, prob_e, 0.0)
                part = part + gate * y[:, le * d_h:(le + 1) * d_h]
            return part

        def make_rdma(step, half, k):
            return pltpu.make_async_remote_copy(
                src_ref=send_buf.at[step, half],
                dst_ref=comm_ref.at[step, half],
                send_sem=send_sems.at[step, half],
                recv_sem=recv_sems.at[step, half],
                device_id=(me ^ k,),
                device_id_type=pl.DeviceIdType.MESH,
            )

        import os
        skip_comm = os.environ.get("SKIP_COMM") == "1"

        acc = [None, None]
        acc[0] = expert_partial(0)
        send_buf[0, 0] = acc[0].astype(jnp.bfloat16)
        rdmas = [[None] * 2 for _ in STEPS]
        if not skip_barrier:
            pl.semaphore_wait(barrier, len(STEPS))
        if not skip_comm:
            rdmas[0][0] = make_rdma(0, 0, STEPS[0])
            rdmas[0][0].start()

        acc[1] = expert_partial(hm)
        send_buf[0, 1] = acc[1].astype(jnp.bfloat16)
        if not skip_comm:
            rdmas[0][1] = make_rdma(0, 1, STEPS[0])
            rdmas[0][1].start()

        shared = jnp.dot(x_bf, sw_ref[:, :].astype(jnp.bfloat16),
                         preferred_element_type=jnp.float32)

        if skip_comm:
            out_ref[0:hm, :] = acc[0] + shared[0:hm]
            out_ref[hm:n_tok, :] = acc[1] + shared[hm:n_tok]
            return

        for step, k in enumerate(STEPS):
            for half in range(2):
                rdmas[step][half].wait()
                acc[half] = acc[half] + comm_ref[step, half].astype(jnp.float32)
                if step + 1 < len(STEPS):
                    send_buf[step + 1, half] = acc[half].astype(jnp.bfloat16)
                    rdmas[step + 1][half] = make_rdma(step + 1, half,
                                                      STEPS[step + 1])
                    rdmas[step + 1][half].start()

        out_ref[0:hm, :] = acc[0] + shared[0:hm]
        out_ref[hm:n_tok, :] = acc[1] + shared[hm:n_tok]

    return pl.pallas_call(
        body,
        out_shape=jax.ShapeDtypeStruct((n_tok, d_h), jnp.float32),
        in_specs=[pl.BlockSpec(memory_space=pltpu.VMEM)] * 5,
        out_specs=pl.BlockSpec(memory_space=pltpu.VMEM),
        scratch_shapes=[
            pltpu.VMEM((len(STEPS), 2, n_tok // 2, d_h), jnp.bfloat16),
            pltpu.VMEM((len(STEPS), 2, n_tok // 2, d_h), jnp.bfloat16),
            pltpu.SemaphoreType.DMA((len(STEPS), 2)),
            pltpu.SemaphoreType.DMA((len(STEPS), 2)),
        ],
        compiler_params=(
            pltpu.CompilerParams()
            if __import__("os").environ.get("SKIP_BARRIER") == "1"
            else pltpu.CompilerParams(collective_id=0)
        ),
    )(x, router_W, route_idx, expert_W, shared_W)
